# Initial kernel scaffold; baseline (speedup 1.0000x reference)
import jax
import jax.numpy as jnp
from jax.experimental import pallas as pl


def kernel(G, edge_index, edge_weight, edge_attribute, problem, W_emb1, b_emb1, W_emb2, b_emb2, Wl1, Wr1, We1, att1, bc1, Wl2, Wr2, We2, att2, bc2, Wg1, bg1, Wg2, bg2, Wu, bu, Wu2, bu2, Wu3, bu3):
    raise NotImplementedError("write your pallas kernel here")


# speedup vs baseline: 8.8264x; 8.8264x over previous
import functools

import jax
import jax.numpy as jnp
from jax import lax
from jax.experimental import pallas as pl
from jax.experimental.pallas import tpu as pltpu
from jax.experimental.pallas import tpu_sc as plsc

N = 10000
E = 320000
DF = 128
H = 64
ED = 21

NPAD = 10240
NW = 32
EW = E // NW
CHK = 80
BLK = 80
NCH = BLK // CHK
NBLK = EW // BLK
GRP = BLK // 16
RPT = NPAD // 16
DEN_R = NPAD // 64
DEN_C = 64

_F32 = jnp.float32


def _dot(a, b):
    return jnp.dot(a, b, preferred_element_type=_F32)


def _node_pre_body(g_ref, w1_ref, b1_ref, w2_ref, b2_ref, wl_ref, wr_ref,
                   xl_ref, xr_ref):
    h = jnp.maximum(_dot(g_ref[...], w1_ref[...]) + b1_ref[...], 0.0)
    h = jnp.maximum(_dot(h, w2_ref[...]) + b2_ref[...], 0.0)
    xl_ref[...] = _dot(h, wl_ref[...])
    xr_ref[...] = _dot(h, wr_ref[...])


def _node_pre(G, W1, b1, W2, b2, Wl, Wr):
    return pl.pallas_call(
        _node_pre_body,
        out_shape=[jax.ShapeDtypeStruct((N, H), _F32),
                   jax.ShapeDtypeStruct((N, H), _F32)],
    )(G, W1, b1, W2, b2, Wl, Wr)


_EB = 3200


def _edge_proj_body(ea_ref, we1_ref, we2_ref, ew1_ref, ew2_ref):
    ea = ea_ref[...]
    ew1_ref[...] = _dot(ea, we1_ref[...])
    ew2_ref[...] = _dot(ea, we2_ref[...])


def _edge_proj(ea, We1, We2):
    return pl.pallas_call(
        _edge_proj_body,
        grid=(E // _EB,),
        in_specs=[
            pl.BlockSpec((_EB, ED), lambda i: (i, 0)),
            pl.BlockSpec((ED, H), lambda i: (0, 0)),
            pl.BlockSpec((ED, H), lambda i: (0, 0)),
        ],
        out_specs=[
            pl.BlockSpec((_EB, H), lambda i: (i, 0)),
            pl.BlockSpec((_EB, H), lambda i: (i, 0)),
        ],
        out_shape=[jax.ShapeDtypeStruct((E, H), _F32),
                   jax.ShapeDtypeStruct((E, H), _F32)],
    )(ea, We1, We2)


def _node_mid_body(s_ref, den_ref, bc_ref, wl_ref, wr_ref, xl_ref, xr_ref):
    s = s_ref[0] + s_ref[1]
    den = jnp.sum(den_ref[...], axis=0)
    h = jnp.maximum(s / (den[:, None] + 1e-16) + bc_ref[...], 0.0)
    xl_ref[...] = _dot(h, wl_ref[...])
    xr_ref[...] = _dot(h, wr_ref[...])


def _node_mid(s, den, bc, Wl, Wr):
    return pl.pallas_call(
        _node_mid_body,
        out_shape=[jax.ShapeDtypeStruct((NPAD, H), _F32),
                   jax.ShapeDtypeStruct((NPAD, H), _F32)],
    )(s, den, bc, Wl, Wr)


def _node_post_body(s_ref, den_ref, bc_ref, wg1_ref, bg1_ref, wg2_ref,
                    bg2_ref, wut_ref, wub_ref, bu_ref, wu2_ref, bu2_ref,
                    wu3_ref, bu3_ref, u_ref):
    s = s_ref[0] + s_ref[1]
    den = jnp.sum(den_ref[...], axis=0)
    h = jnp.maximum(s / (den[:, None] + 1e-16) + bc_ref[...], 0.0)
    h = jnp.maximum(_dot(h, wg1_ref[...]) + bg1_ref[...], 0.0)
    h = _dot(h, wg2_ref[...]) + bg2_ref[...]
    rowid = lax.broadcasted_iota(jnp.int32, (NPAD, 1), 0)
    hm = jnp.where(rowid < N, h, 0.0)
    gmean = jnp.sum(hm, axis=0, keepdims=True) * (1.0 / N)
    u = jnp.maximum(_dot(h, wut_ref[...]) + _dot(gmean, wub_ref[...])
                    + bu_ref[...], 0.0)
    u = jnp.maximum(_dot(u, wu2_ref[...]) + bu2_ref[...], 0.0)
    u_ref[...] = _dot(u, wu3_ref[...]) + bu3_ref[...]


def _node_post(s, den, bc, Wg1, bg1, Wg2, bg2, Wut, Wub, bu, Wu2, bu2,
               Wu3, bu3):
    return pl.pallas_call(
        _node_post_body,
        out_shape=jax.ShapeDtypeStruct((NPAD, 1), _F32),
    )(s, den, bc, Wg1, bg1, Wg2, bg2, Wut, Wub, bu, Wu2, bu2, Wu3, bu3)


def _sc_edge_body(src_ref, dst_ref, xl_ref, xr_ref, ew_ref, att_ref,
                  s_out, den_out,
                  src_l, dst_l, xlb, xrb, ewb, att_v, tbuf, den_l, s_sh,
                  sem_g, sem_s):
    cc = lax.axis_index("c")
    sid = lax.axis_index("s")
    wid = sid * 2 + cc

    zero16 = jnp.zeros((16,), _F32)

    def zero_body(r, carry):
        for k in range(4):
            den_l[r, pl.ds(k * 16, 16)] = zero16
        return carry

    lax.fori_loop(0, DEN_R, zero_body, 0)
    for k in range(4):
        pltpu.sync_copy(den_l, s_sh.at[pl.ds(sid * RPT + k * DEN_R, DEN_R)])
    plsc.subcore_barrier()

    pltpu.sync_copy(att_ref, att_v)
    attv = [att_v[pl.ds(k * 16, 16)] for k in range(4)]

    def block_body(j, carry):
        ibase = wid * (EW // CHK) + j * NCH
        pltpu.sync_copy(src_ref.at[pl.ds(ibase, NCH)], src_l)
        pltpu.sync_copy(dst_ref.at[pl.ds(ibase, NCH)], dst_l)
        ebase = wid * EW + j * BLK
        cps = [pltpu.async_copy(ew_ref.at[pl.ds(ebase, BLK)], ewb, sem_g)]
        for ch in range(NCH):
            cps.append(pltpu.async_copy(
                xl_ref.at[src_l.at[ch]],
                xlb.at[pl.ds(ch * CHK, CHK)], sem_g))
            cps.append(pltpu.async_copy(
                xr_ref.at[dst_l.at[ch]],
                xrb.at[pl.ds(ch * CHK, CHK)], sem_g))
        for cp in cps:
            cp.wait()

        def group_body(g, gcarry):
            ch = g // (CHK // 16)
            off = (g % (CHK // 16)) * 16
            dstv = dst_l[ch, pl.ds(off, 16)]
            base_r = g * 16
            for i in range(16):
                r = base_r + i
                acc = None
                for k in range(4):
                    ds = pl.ds(k * 16, 16)
                    sxy = xlb[r, ds] + xrb[r, ds] + ewb[r, ds]
                    lr = jnp.maximum(sxy, 0.2 * sxy)
                    t = lr * attv[k]
                    acc = t if acc is None else acc + t
                tbuf[pl.ds(i * 16, 16)] = acc
            lane16 = lax.iota(jnp.int32, 16) * 16
            lv = None
            for l in range(16):
                col = plsc.load_gather(tbuf, [lane16 + l])
                lv = col if lv is None else lv + col
            exv = jnp.exp(lv)
            rowv = lax.shift_right_arithmetic(dstv, 6)
            colv = jnp.bitwise_and(dstv, 63)
            plsc.addupdate_scatter(den_l, [rowv, colv], exv)
            for i in range(16):
                r = base_r + i
                exs = exv[i]
                for k in range(4):
                    ds = pl.ds(k * 16, 16)
                    xlb[r, ds] = xlb[r, ds] * exs
            return gcarry

        lax.fori_loop(0, GRP, group_body, 0)
        scs = [pltpu.async_copy(xlb.at[pl.ds(ch * CHK, CHK)],
                                s_sh.at[dst_l.at[ch]], sem_s, add=True)
               for ch in range(NCH)]
        for cp in scs:
            cp.wait()
        return carry

    lax.fori_loop(0, NBLK, block_body, 0)
    plsc.subcore_barrier()

    pltpu.sync_copy(den_l, den_out.at[wid])
    for k in range(4):
        off = sid * RPT + k * DEN_R
        pltpu.sync_copy(s_sh.at[pl.ds(off, DEN_R)],
                        s_out.at[cc, pl.ds(off, DEN_R)])


_edge_pass = functools.partial(
    pl.kernel,
    out_type=[jax.ShapeDtypeStruct((2, NPAD, H), _F32),
              jax.ShapeDtypeStruct((NW, DEN_R, DEN_C), _F32)],
    mesh=plsc.VectorSubcoreMesh(core_axis_name="c", subcore_axis_name="s"),
    compiler_params=pltpu.CompilerParams(needs_layout_passes=False,
                                         use_tc_tiling_on_sc=False),
    scratch_types=[
        pltpu.VMEM((NCH, CHK), jnp.int32),
        pltpu.VMEM((NCH, CHK), jnp.int32),
        pltpu.VMEM((BLK, H), _F32),
        pltpu.VMEM((BLK, H), _F32),
        pltpu.VMEM((BLK, H), _F32),
        pltpu.VMEM((H,), _F32),
        pltpu.VMEM((256,), _F32),
        pltpu.VMEM((DEN_R, DEN_C), _F32),
        pltpu.VMEM_SHARED((NPAD, H), _F32),
        pltpu.SemaphoreType.DMA,
        pltpu.SemaphoreType.DMA,
    ],
)(_sc_edge_body)


def kernel(G, edge_index, edge_weight, edge_attribute, problem, W_emb1,
           b_emb1, W_emb2, b_emb2, Wl1, Wr1, We1, att1, bc1, Wl2, Wr2, We2,
           att2, bc2, Wg1, bg1, Wg2, bg2, Wu, bu, Wu2, bu2, Wu3, bu3):
    src2 = edge_index[0].reshape(E // CHK, CHK)
    dst2 = edge_index[1].reshape(E // CHK, CHK)

    xl1, xr1 = _node_pre(G, W_emb1, b_emb1.reshape(1, H), W_emb2,
                         b_emb2.reshape(1, H), Wl1, Wr1)
    ew1, ew2 = _edge_proj(edge_attribute, We1, We2)

    s1, den1 = _edge_pass(src2, dst2, xl1, xr1, ew1, att1)
    xl2, xr2 = _node_mid(s1, den1.reshape(NW, NPAD), bc1.reshape(1, H),
                         Wl2, Wr2)

    s2, den2 = _edge_pass(src2, dst2, xl2, xr2, ew2, att2)
    u = _node_post(s2, den2.reshape(NW, NPAD), bc2.reshape(1, H), Wg1,
                   bg1.reshape(1, H), Wg2, bg2.reshape(1, H), Wu[:H],
                   Wu[H:], bu.reshape(1, H), Wu2, bu2.reshape(1, H), Wu3,
                   bu3.reshape(1, 1))
    return u[:N]
